# baseline (speedup 1.0000x reference)
import functools

import jax
import jax.numpy as jnp
from jax import lax
from jax.experimental import pallas as pl
from jax.experimental.pallas import tpu as pltpu
from jax.experimental.pallas import tpu_sc as plsc

_V = 1000
_D = 200
_DP = 256
_B = 1024
_L = 200
_NC = 2
_NS = 16
_NW = _NC * _NS
_MB = _B // 128
_NBLK = _L * _MB
_PER_W = _NBLK // _NW
_JB = 13

_mesh = plsc.VectorSubcoreMesh(core_axis_name="c", subcore_axis_name="s")


@functools.partial(
    pl.kernel,
    mesh=_mesh,
    out_type=jax.ShapeDtypeStruct((_L * _D, _B), jnp.float32),
    compiler_params=pltpu.CompilerParams(
        use_tc_tiling_on_sc=True, needs_layout_passes=False),
    scratch_types=(
        [pltpu.VMEM((_PER_W, 128), jnp.int32)]
        + [pltpu.VMEM((128, _DP), jnp.float32) for _ in range(2)]
        + [pltpu.VMEM((16 * _JB, 128), jnp.float32) for _ in range(2)]
        + [pltpu.SemaphoreType.DMA for _ in range(4)]
    ),
)
def _embed(idx_hbm, table_hbm, out_hbm, idx_v, r0, r1, t0, t1, g0, g1, s0, s1):
    rows = (r0, r1)
    tbuf = (t0, t1)
    gsem = (g0, g1)
    ssem = (s0, s1)

    sid = lax.axis_index("s")
    wid = sid * _NC + lax.axis_index("c")

    pltpu.sync_copy(idx_hbm.at[wid], idx_v)

    def gather(i, p):
        return pltpu.make_async_copy(
            table_hbm.at[idx_v.at[i]], rows[p], gsem[p])

    def scat(i, p):
        blk = wid * _PER_W + i
        l = blk // _MB
        m = blk - l * _MB
        return pltpu.make_async_copy(
            tbuf[p].at[pl.ds(0, _D), :],
            out_hbm.at[pl.ds(l * _D, _D), pl.ds(m * 128, 128)],
            ssem[p])

    iota = jnp.arange(16, dtype=jnp.int32)
    jvecs = [u * 16 + iota for u in range(_JB)]

    def transpose_block(p):
        r = rows[p]
        t = tbuf[p]

        @plsc.parallel_loop(0, 128, 1, unroll=4)
        def _(bb):
            bvec = iota * 0 + bb
            for u in range(_JB):
                plsc.store_scatter(t, [jvecs[u], bvec],
                                   r[bb, pl.ds(u * 16, 16)])

    gather(0, 0).start()

    def pair_body(q, carry):
        i0 = 2 * q
        gather(i0, 0).wait()
        pl.when(q > 0)(lambda: scat(i0 - 2, 0).wait())
        gather(i0 + 1, 1).start()
        transpose_block(0)
        scat(i0, 0).start()
        gather(i0 + 1, 1).wait()
        pl.when(q > 0)(lambda: scat(i0 - 1, 1).wait())
        pl.when(q < _PER_W // 2 - 1)(lambda: gather(i0 + 2, 0).start())
        transpose_block(1)
        scat(i0 + 1, 1).start()
        return carry

    lax.fori_loop(0, _PER_W // 2, pair_body, 0)

    scat(_PER_W - 2, 0).wait()
    scat(_PER_W - 1, 1).wait()


def kernel(inputs, table):
    idx = inputs.reshape(_MB, 128, _L).transpose(2, 0, 1).reshape(_NW, _PER_W, 128)
    table_p = jnp.pad(table, ((0, 0), (0, _DP - _D)))
    out = _embed(idx, table_p)
    return out.T


# speedup vs baseline: 3.0843x; 3.0843x over previous
import functools

import jax
import jax.numpy as jnp
from jax import lax
from jax.experimental import pallas as pl
from jax.experimental.pallas import tpu as pltpu
from jax.experimental.pallas import tpu_sc as plsc

_V = 1000
_D = 200
_DP = 256
_B = 1024
_L = 200
_NC = 2
_NS = 16
_NW = _NC * _NS
_MB = _B // 128
_NBLK = _L * _MB
_PER_W = _NBLK // _NW
_JB = 13

_mesh = plsc.VectorSubcoreMesh(core_axis_name="c", subcore_axis_name="s")


@functools.partial(
    pl.kernel,
    mesh=_mesh,
    out_type=jax.ShapeDtypeStruct((_L * _D, _B), jnp.float32),
    compiler_params=pltpu.CompilerParams(
        use_tc_tiling_on_sc=True, needs_layout_passes=False),
    scratch_types=(
        [pltpu.VMEM((_PER_W, 128), jnp.int32)]
        + [pltpu.VMEM((128, _DP), jnp.float32) for _ in range(2)]
        + [pltpu.VMEM((16 * _JB, 128), jnp.float32) for _ in range(2)]
        + [pltpu.SemaphoreType.DMA for _ in range(4)]
    ),
)
def _embed(idx_hbm, table_hbm, out_hbm, idx_v, r0, r1, t0, t1, g0, g1, s0, s1):
    rows = (r0, r1)
    tbuf = (t0, t1)
    gsem = (g0, g1)
    ssem = (s0, s1)

    sid = lax.axis_index("s")
    wid = sid * _NC + lax.axis_index("c")

    pltpu.sync_copy(idx_hbm.at[wid], idx_v)

    def gather(i, p):
        return pltpu.make_async_copy(
            table_hbm.at[idx_v.at[i]], rows[p], gsem[p])

    def scat(i, p):
        blk = wid * _PER_W + i
        l = blk // _MB
        m = blk - l * _MB
        return pltpu.make_async_copy(
            tbuf[p].at[pl.ds(0, _D), :],
            out_hbm.at[pl.ds(l * _D, _D), pl.ds(m * 128, 128)],
            ssem[p])

    iota = jnp.arange(16, dtype=jnp.int32)
    rots = [jnp.remainder(iota + d, 16) for d in range(16)]

    def transpose_block(p):
        r = rows[p]
        t = tbuf[p]

        @plsc.parallel_loop(0, _JB * 8, 1)
        def _(v):
            u = v // 8
            tt = v - u * 8
            jvec = u * 16 + iota
            for d in range(16):
                bvec = tt * 16 + rots[d]
                vals = plsc.load_gather(r, [bvec, jvec])
                plsc.store_scatter(t, [jvec, bvec], vals)

    gather(0, 0).start()

    def pair_body(q, carry):
        i0 = 2 * q
        gather(i0, 0).wait()
        pl.when(q > 0)(lambda: scat(i0 - 2, 0).wait())
        gather(i0 + 1, 1).start()
        transpose_block(0)
        scat(i0, 0).start()
        gather(i0 + 1, 1).wait()
        pl.when(q > 0)(lambda: scat(i0 - 1, 1).wait())
        pl.when(q < _PER_W // 2 - 1)(lambda: gather(i0 + 2, 0).start())
        transpose_block(1)
        scat(i0 + 1, 1).start()
        return carry

    lax.fori_loop(0, _PER_W // 2, pair_body, 0)

    scat(_PER_W - 2, 0).wait()
    scat(_PER_W - 1, 1).wait()


def kernel(inputs, table):
    idx = inputs.reshape(_MB, 128, _L).transpose(2, 0, 1).reshape(_NW, _PER_W, 128)
    table_p = jnp.pad(table, ((0, 0), (0, _DP - _D)))
    out = _embed(idx, table_p)
    return out.T
